# baseline (speedup 1.0000x reference)
import jax
import jax.numpy as jnp
from jax import lax
from jax.experimental import pallas as pl
from jax.experimental.pallas import tpu as pltpu
from jax.experimental.pallas import tpu_sc as plsc

NC = 2
NS = 16
NW = NC * NS

B = 16384
E = 64
C = 6
BPW = B // NW
CHUNK = 128
NCHUNK = BPW // CHUNK


def _sc_body(tgt_hbm, ctx_hbm, ttab_hbm, ctab_hbm, out_hbm, *scratch):
    tgt_idx = scratch[0:NCHUNK]
    ctx_idx = scratch[NCHUNK:2 * NCHUNK]
    wrows = scratch[2 * NCHUNK:2 * NCHUNK + 2]
    crows = scratch[2 * NCHUNK + 2:2 * NCHUNK + 4]
    outv = scratch[2 * NCHUNK + 4:2 * NCHUNK + 6]
    semI = scratch[2 * NCHUNK + 6:3 * NCHUNK + 6]
    semW = scratch[3 * NCHUNK + 6:3 * NCHUNK + 8]
    semC = scratch[3 * NCHUNK + 8:3 * NCHUNK + 10]

    wid = lax.axis_index("s") * NC + lax.axis_index("c")
    base = wid * BPW

    idx_handles = []
    for k in range(NCHUNK):
        h1 = pltpu.async_copy(
            tgt_hbm.at[pl.ds(base + k * CHUNK, CHUNK)], tgt_idx[k], semI[k])
        h2 = pltpu.async_copy(
            ctx_hbm.at[pl.ds((base + k * CHUNK) * C, CHUNK * C)], ctx_idx[k],
            semI[k])
        idx_handles.append((h1, h2))

    gather_handles = [None, None]

    def start_gathers(k):
        s = k % 2
        idx_handles[k][0].wait()
        idx_handles[k][1].wait()
        hw = pltpu.async_copy(ttab_hbm.at[tgt_idx[k]], wrows[s], semW[s])
        hcs = []
        for j in range(C):
            hcs.append(pltpu.async_copy(
                ctab_hbm.at[ctx_idx[k].at[pl.ds(j * CHUNK, CHUNK)]],
                crows[s].at[pl.ds(j * CHUNK, CHUNK), :], semC[s]))
        gather_handles[s] = (hw, hcs)

    def compute_chunk(w_ref, c_ref, o_ref):
        lane = lax.iota(jnp.int32, 16)
        lmask = lane < C

        def row(i, carry):
            w0 = w_ref[i, pl.ds(0, 16)]
            w1 = w_ref[i, pl.ds(16, 16)]
            w2 = w_ref[i, pl.ds(32, 16)]
            w3 = w_ref[i, pl.ds(48, 16)]
            res = jnp.zeros((16,), jnp.float32)
            for c in range(C):
                r = i * C + c
                p = (w0 * c_ref[r, pl.ds(0, 16)]
                     + w1 * c_ref[r, pl.ds(16, 16)]
                     + w2 * c_ref[r, pl.ds(32, 16)]
                     + w3 * c_ref[r, pl.ds(48, 16)])
                res = jnp.where(lane == c, jnp.sum(p), res)
            plsc.store_scatter(o_ref, [i * C + lane], res, mask=lmask)
            return carry
        lax.fori_loop(0, CHUNK, row, 0)

    start_gathers(0)
    for k in range(NCHUNK):
        if k + 1 < NCHUNK:
            start_gathers(k + 1)
        s = k % 2
        gather_handles[s][0].wait()
        for h in gather_handles[s][1]:
            h.wait()
        compute_chunk(wrows[s], crows[s], outv[s])
        pltpu.sync_copy(
            outv[s], out_hbm.at[pl.ds((base + k * CHUNK) * C, CHUNK * C)])


_mesh = plsc.VectorSubcoreMesh(core_axis_name="c", subcore_axis_name="s")

_scratch_types = (
    [pltpu.VMEM((CHUNK,), jnp.int32) for _ in range(NCHUNK)]
    + [pltpu.VMEM((CHUNK * C,), jnp.int32) for _ in range(NCHUNK)]
    + [pltpu.VMEM((CHUNK, E), jnp.float32) for _ in range(2)]
    + [pltpu.VMEM((CHUNK * C, E), jnp.float32) for _ in range(2)]
    + [pltpu.VMEM((CHUNK * C,), jnp.float32) for _ in range(2)]
    + [pltpu.SemaphoreType.DMA for _ in range(NCHUNK + 4)]
)

_sc_call = pl.kernel(
    _sc_body,
    out_type=jax.ShapeDtypeStruct((B * C,), jnp.float32),
    mesh=_mesh,
    scratch_types=_scratch_types,
    compiler_params=pltpu.CompilerParams(
        needs_layout_passes=False, use_tc_tiling_on_sc=False),
)


def kernel(target, context, target_table, context_table):
    tgt = target.astype(jnp.int32)
    ctx = context.astype(jnp.int32).reshape(B * C)
    out_flat = _sc_call(tgt, ctx, target_table, context_table)
    return out_flat.reshape(B, C)
